# Initial kernel scaffold; baseline (speedup 1.0000x reference)
import jax
import jax.numpy as jnp
from jax.experimental import pallas as pl


def kernel(inputs, token_table, pos_table):
    raise NotImplementedError("write your pallas kernel here")


# speedup vs baseline: 4.1509x; 4.1509x over previous
import functools

import jax
import jax.numpy as jnp
from jax import lax
from jax.experimental import pallas as pl
from jax.experimental.pallas import tpu as pltpu
from jax.experimental.pallas import tpu_sc as plsc


def _build_kernel(N, V, L, D):
    info = plsc.get_sparse_core_info()
    NC, NS = info.num_cores, info.num_subcores
    NW = NC * NS
    per_w = N // NW
    CS = 4
    C = CS * L
    G = per_w // C
    NV = D // 16
    assert N % NW == 0 and per_w % C == 0 and G % 2 == 0 and D % 16 == 0

    subs = []
    o = 0
    while o < C:
        n = min(128, C - o)
        subs.append((o, n))
        o += n

    mesh = plsc.VectorSubcoreMesh(core_axis_name="c", subcore_axis_name="s")

    @functools.partial(
        pl.kernel,
        mesh=mesh,
        out_type=jax.ShapeDtypeStruct((N, D), jnp.float32),
        compiler_params=pltpu.CompilerParams(use_tc_tiling_on_sc=False),
        scratch_types=[
            pltpu.VMEM((C,), jnp.int32),
            pltpu.VMEM((C,), jnp.int32),
            pltpu.VMEM((C, D), jnp.float32),
            pltpu.VMEM((C, D), jnp.float32),
            pltpu.VMEM((L, D), jnp.float32),
            pltpu.SemaphoreType.DMA,
            pltpu.SemaphoreType.DMA,
            pltpu.SemaphoreType.DMA,
            pltpu.SemaphoreType.DMA,
        ],
    )
    def k(idx_hbm, tab_hbm, pos_hbm, out_hbm,
          idx0, idx1, rows0, rows1, pos_v, gsem0, gsem1, ssem0, ssem1):
        wid = lax.axis_index("s") * NC + lax.axis_index("c")
        base = wid * per_w

        pltpu.sync_copy(pos_hbm, pos_v)

        idx_bufs = (idx0, idx1)
        rows_bufs = (rows0, rows1)
        gsems = (gsem0, gsem1)
        ssems = (ssem0, ssem1)

        def start_chunk(g, b):
            row0 = base + g * C
            pltpu.sync_copy(idx_hbm.at[pl.ds(row0, C)], idx_bufs[b])
            handles = []
            for (o, n) in subs:
                handles.append(pltpu.async_copy(
                    tab_hbm.at[idx_bufs[b].at[pl.ds(o, n)]],
                    rows_bufs[b].at[pl.ds(o, n)],
                    gsems[b]))
            return handles

        def add_pos(b):
            rows = rows_bufs[b]

            def body(l, _):
                for d in range(NV):
                    pv = pos_v[l, pl.ds(16 * d, 16)]
                    for s in range(CS):
                        r = s * L + l
                        rows[r, pl.ds(16 * d, 16)] += pv
                return 0

            lax.fori_loop(0, L, body, 0)

        def start_scatter(g, b):
            row0 = base + g * C
            return pltpu.async_copy(
                rows_bufs[b], out_hbm.at[pl.ds(row0, C)], ssems[b])

        def wait_scatter(g, b):
            row0 = base + g * C
            pltpu.make_async_copy(
                rows_bufs[b], out_hbm.at[pl.ds(row0, C)], ssems[b]).wait()

        def pair(p, _):
            g0 = 2 * p
            g1 = g0 + 1

            @pl.when(p > 0)
            def _():
                wait_scatter(g0 - 2, 0)

            h0 = start_chunk(g0, 0)

            @pl.when(p > 0)
            def _():
                wait_scatter(g1 - 2, 1)

            h1 = start_chunk(g1, 1)

            for h in h0:
                h.wait()
            add_pos(0)
            start_scatter(g0, 0)

            for h in h1:
                h.wait()
            add_pos(1)
            start_scatter(g1, 1)
            return 0

        lax.fori_loop(0, G // 2, pair, 0)
        wait_scatter(G - 2, 0)
        wait_scatter(G - 1, 1)

    return k


def kernel(inputs, token_table, pos_table):
    B, L = inputs.shape
    V, D = token_table.shape
    N = B * L
    idx_flat = inputs.reshape(N).astype(jnp.int32)
    k = _build_kernel(N, V, L, D)
    out = k(idx_flat, token_table, pos_table)
    return out.reshape(B, L, D)
